# Initial kernel scaffold; baseline (speedup 1.0000x reference)
import jax
import jax.numpy as jnp
from jax.experimental import pallas as pl


def kernel(coords, factors, weights, bias):
    raise NotImplementedError("write your pallas kernel here")


# speedup vs baseline: 1.0905x; 1.0905x over previous
import functools

import jax
import jax.numpy as jnp
from jax import lax
from jax.experimental import pallas as pl
from jax.experimental.pallas import tpu as pltpu
from jax.experimental.pallas import tpu_sc as plsc

NC = 2
NS = 16
NW = NC * NS
LANES = 16
CHUNK = 128


@functools.lru_cache(maxsize=None)
def _build(B, H, V, R):
    BPW = B // NW
    NCH = BPW // CHUNK
    NIDX = H * NCH

    mesh = plsc.VectorSubcoreMesh(core_axis_name="c", subcore_axis_name="s")

    @functools.partial(
        pl.kernel,
        mesh=mesh,
        out_type=jax.ShapeDtypeStruct((B,), jnp.float32),
        scratch_types=[
            pltpu.VMEM((NIDX, CHUNK), jnp.int32),
            pltpu.VMEM((2, BPW, R), jnp.float32),
            pltpu.VMEM((BPW, R), jnp.float32),
            pltpu.VMEM((BPW,), jnp.float32),
            pltpu.VMEM((3, LANES), jnp.float32),
            pltpu.SemaphoreType.DMA,
            pltpu.SemaphoreType.DMA,
        ],
        compiler_params=pltpu.CompilerParams(
            needs_layout_passes=False, use_tc_tiling_on_sc=False),
    )
    def cp_kernel(factors_hbm, idx_hbm, wb_hbm, out_hbm,
                  idx_v, buf, prod, outv, wb_v, sem0, sem1):
        cid = lax.axis_index("c")
        sid = lax.axis_index("s")
        wid = cid * NS + sid
        sems = (sem0, sem1)

        pltpu.sync_copy(idx_hbm.at[pl.ds(wid * NIDX, NIDX)], idx_v)
        pltpu.sync_copy(wb_hbm, wb_v)
        w0 = wb_v[0]
        w1 = wb_v[1]
        bias_vec = wb_v[2]

        def fire(m, slot):
            descs = []
            for c in range(NCH):
                descs.append(pltpu.async_copy(
                    factors_hbm.at[idx_v.at[m * NCH + c]],
                    buf.at[slot, pl.ds(c * CHUNK, CHUNK)],
                    sems[slot]))
            return descs

        pending = {0: fire(0, 0)}
        for m in range(H):
            slot = m % 2
            if m + 1 < H:
                pending[(m + 1) % 2] = fire(m + 1, (m + 1) % 2)
            for d in pending[slot]:
                d.wait()
            if m == 0:
                def body0(i, _):
                    prod[i, pl.ds(0, LANES)] = (
                        buf[slot, i, pl.ds(0, LANES)] * w0)
                    prod[i, pl.ds(LANES, LANES)] = (
                        buf[slot, i, pl.ds(LANES, LANES)] * w1)
                    return 0
                lax.fori_loop(0, BPW, body0, 0)
            else:
                def bodym(i, _):
                    prod[i, pl.ds(0, LANES)] = (
                        prod[i, pl.ds(0, LANES)]
                        * buf[slot, i, pl.ds(0, LANES)])
                    prod[i, pl.ds(LANES, LANES)] = (
                        prod[i, pl.ds(LANES, LANES)]
                        * buf[slot, i, pl.ds(LANES, LANES)])
                    return 0
                lax.fori_loop(0, BPW, bodym, 0)

        iota = lax.broadcasted_iota(jnp.int32, (LANES,), 0)

        def red_body(g, _):
            rows = g * LANES + iota
            acc = bias_vec
            for j in range(R):
                col = jnp.full((LANES,), j, dtype=jnp.int32)
                acc = acc + plsc.load_gather(prod, [rows, col])
            outv[pl.ds(g * LANES, LANES)] = acc
            return 0
        lax.fori_loop(0, BPW // LANES, red_body, 0)

        pltpu.sync_copy(outv, out_hbm.at[pl.ds(wid * BPW, BPW)])

    return cp_kernel


def kernel(coords, factors, weights, bias):
    H, V, R = factors.shape
    B = coords.shape[0]
    cp_kernel = _build(B, H, V, R)

    flat_idx = coords.astype(jnp.int32) + jnp.arange(
        H, dtype=jnp.int32)[None, :] * V
    BPW = B // NW
    NCH = BPW // CHUNK
    idx_arr = (flat_idx.reshape(NW, NCH, CHUNK, H)
               .transpose(0, 3, 1, 2)
               .reshape(NW * H * NCH, CHUNK))

    wb = jnp.concatenate([
        weights.astype(jnp.float32),
        jnp.broadcast_to(bias.astype(jnp.float32), (LANES,)),
    ]).reshape(3, LANES)

    factors2d = factors.reshape(H * V, R)
    return cp_kernel(factors2d, idx_arr, wb)
